# baseline (speedup 1.0000x reference)
import jax
import jax.numpy as jnp
from jax.experimental import pallas as pl
from jax.experimental.pallas import tpu as pltpu

_R = 64
_C = 16384
_NBUF = 16
_BLKC = _C // _NBUF


def _copy_body(x_ref, o_ref, bufs, in_sems, out_sems):
    for j in range(_NBUF):
        pltpu.make_async_copy(
            x_ref.at[:, pl.ds(j * _BLKC, _BLKC)], bufs.at[j], in_sems.at[j]
        ).start()
    for j in range(_NBUF):
        pltpu.make_async_copy(
            x_ref.at[:, pl.ds(j * _BLKC, _BLKC)], bufs.at[j], in_sems.at[j]
        ).wait()
        pltpu.make_async_copy(
            bufs.at[j], o_ref.at[:, pl.ds(j * _BLKC, _BLKC)], out_sems.at[j]
        ).start()
    for j in range(_NBUF):
        pltpu.make_async_copy(
            bufs.at[j], o_ref.at[:, pl.ds(j * _BLKC, _BLKC)], out_sems.at[j]
        ).wait()


def kernel(x, memoryData, memoryTarget):
    xt = x.T
    out = pl.pallas_call(
        _copy_body,
        out_shape=jax.ShapeDtypeStruct((_R, _C), jnp.float32),
        in_specs=[pl.BlockSpec(memory_space=pl.ANY)],
        out_specs=pl.BlockSpec(memory_space=pl.ANY),
        scratch_shapes=[
            pltpu.VMEM((_NBUF, _R, _BLKC), jnp.float32),
            pltpu.SemaphoreType.DMA((_NBUF,)),
            pltpu.SemaphoreType.DMA((_NBUF,)),
        ],
    )(xt)
    return out.T


# speedup vs baseline: 1.0628x; 1.0628x over previous
import jax
import jax.numpy as jnp
from jax.experimental import pallas as pl
from jax.experimental.pallas import tpu as pltpu

_R = 64
_C = 16384
_NBUF = 4
_BLKC = _C // _NBUF


def _copy_body(x_ref, o_ref, bufs, in_sems, out_sems):
    for j in range(_NBUF):
        pltpu.make_async_copy(
            x_ref.at[:, pl.ds(j * _BLKC, _BLKC)], bufs.at[j], in_sems.at[j]
        ).start()
    for j in range(_NBUF):
        pltpu.make_async_copy(
            x_ref.at[:, pl.ds(j * _BLKC, _BLKC)], bufs.at[j], in_sems.at[j]
        ).wait()
        pltpu.make_async_copy(
            bufs.at[j], o_ref.at[:, pl.ds(j * _BLKC, _BLKC)], out_sems.at[j]
        ).start()
    for j in range(_NBUF):
        pltpu.make_async_copy(
            bufs.at[j], o_ref.at[:, pl.ds(j * _BLKC, _BLKC)], out_sems.at[j]
        ).wait()


def kernel(x, memoryData, memoryTarget):
    xt = x.T
    out = pl.pallas_call(
        _copy_body,
        out_shape=jax.ShapeDtypeStruct((_R, _C), jnp.float32),
        in_specs=[pl.BlockSpec(memory_space=pl.ANY)],
        out_specs=pl.BlockSpec(memory_space=pl.ANY),
        scratch_shapes=[
            pltpu.VMEM((_NBUF, _R, _BLKC), jnp.float32),
            pltpu.SemaphoreType.DMA((_NBUF,)),
            pltpu.SemaphoreType.DMA((_NBUF,)),
        ],
    )(xt)
    return out.T
